# baseline (speedup 1.0000x reference)
import jax
import jax.numpy as jnp
from jax import lax
from jax.experimental import pallas as pl
from jax.experimental.pallas import tpu as pltpu
from jax.experimental.pallas import tpu_sc as plsc

_F = 26
_V = 100001
_D = 32
_B = 4096

_NC = 2
_NS = 16
_NW = _NC * _NS
_ROWS = _B * _F
_RPW = _ROWS // _NW
_CHUNK = 128
_NCHUNK = _RPW // _CHUNK


def _gather_body(tab_hbm, idx_hbm, out_hbm, idx_v, rows_v, sem):
    wid = lax.axis_index("s") * _NC + lax.axis_index("c")
    pltpu.sync_copy(idx_hbm.at[wid], idx_v)

    def chunk(j, carry):
        cp = pltpu.async_copy(
            tab_hbm.at[idx_v.at[j]],
            rows_v.at[pl.ds(j * _CHUNK, _CHUNK)],
            sem,
        )
        cp.wait()
        return carry

    lax.fori_loop(0, _NCHUNK, chunk, 0)
    base = wid * _RPW
    pltpu.sync_copy(rows_v, out_hbm.at[pl.ds(base, _RPW)])


@jax.jit
def kernel(indices, tables):
    flat_idx = indices + (jnp.arange(_F, dtype=jnp.int32) * _V)[None, :]
    flat_idx = flat_idx.reshape(_NW, _NCHUNK, _CHUNK)
    tab = tables.reshape(_F * _V, _D)

    gather = pl.kernel(
        _gather_body,
        out_type=jax.ShapeDtypeStruct((_ROWS, _D), jnp.float32),
        mesh=plsc.VectorSubcoreMesh(core_axis_name="c", subcore_axis_name="s"),
        scratch_types=[
            pltpu.VMEM((_NCHUNK, _CHUNK), jnp.int32),
            pltpu.VMEM((_RPW, _D), jnp.float32),
            pltpu.SemaphoreType.DMA,
        ],
        compiler_params=pltpu.CompilerParams(use_tc_tiling_on_sc=False),
    )
    out = gather(tab, flat_idx)
    return out.reshape(_B, _F, _D)


# speedup vs baseline: 24.4756x; 24.4756x over previous
import jax
import jax.numpy as jnp
from jax import lax
from jax.experimental import pallas as pl
from jax.experimental.pallas import tpu as pltpu
from jax.experimental.pallas import tpu_sc as plsc

_F = 26
_V = 100001
_D = 32
_B = 4096

_FPS = 13
_NG = 7
_W = 2176
_NFULL = 45
_WLAST = 45 * _W
_TLO = _WLAST + 2048
_TAIL0 = _V - 128
_CHK = _B // 16
_GW = 2 * _D * _B


def _body(tab, idx4, tailt, out, acc, win, idxv, cmp_v, cmp_p,
          st_v, st_a):
    sc = lax.axis_index("c")
    t = lax.axis_index("s")
    i16 = lax.iota(jnp.int32, 16)

    def bc(x):
        return jnp.full((16,), x, jnp.int32)

    def extract(ncur, base_col, fg, src_ref):
        def do_chunk(c, carry):
            mb0 = c * 32
            for g in range(2):
                mb = mb0 + g * 16
                vvec = cmp_v[pl.ds(mb, 16)]
                pvec = cmp_p[pl.ds(mb, 16)]
                valid = (bc(mb) + i16) < bc(ncur)
                col = jnp.where(valid, vvec - bc(base_col), bc(0))
                for d in range(32):
                    vals = plsc.load_gather(
                        src_ref, [jnp.full((16,), d, jnp.int32), col]
                    )
                    st_v[pl.ds(d * 32 + g * 16, 16)] = vals
                    addr = bc((fg * _D + d) * _B) + pvec
                    addr = jnp.where(
                        valid, addr, jnp.full((16,), -1, jnp.int32)
                    )
                    st_a[pl.ds(d * 32 + g * 16, 16)] = addr
            pltpu.sync_copy(
                st_v, acc.at[plsc.Indices(st_a, ignored_value=-1)]
            )
            return carry

        lax.fori_loop(0, (ncur + 31) // 32, do_chunk, 0)

    def scan(lo, hi):
        def chunk(i, cur):
            ids = idxv[i, :]
            m = (ids >= bc(lo)) & (ids < bc(hi))
            cnt = jnp.sum(m.astype(jnp.int32))

            @pl.when(cnt > 0)
            def _():
                pos = bc(i * 16) + i16
                plsc.store_compressed(cmp_v.at[pl.ds(cur, 16)], ids, mask=m)
                plsc.store_compressed(cmp_p.at[pl.ds(cur, 16)], pos, mask=m)

            return cur + cnt

        return lax.fori_loop(0, _CHK, chunk, 0)

    def field_of_group(g, fg):
        fl = g * 2 + fg
        f = sc * _FPS + fl
        rowb = pl.multiple_of(f * _D, 8)
        pltpu.sync_copy(idx4.at[f], idxv)

        for h in range(2):
            c0 = pl.multiple_of((h * 16 + t) * _W, 128)
            pltpu.sync_copy(tab.at[pl.ds(rowb, _D), pl.ds(c0, _W)], win)
            n = scan(c0, c0 + _W)
            extract(n, c0, fg, win)

        @pl.when(t < 13)
        def _():
            c2 = pl.multiple_of((32 + t) * _W, 128)
            pltpu.sync_copy(tab.at[pl.ds(rowb, _D), pl.ds(c2, _W)], win)
            n = scan(c2, c2 + _W)
            extract(n, c2, fg, win)

        @pl.when(t == 13)
        def _():
            pltpu.sync_copy(
                tab.at[pl.ds(rowb, _D), pl.ds(_WLAST, 2048)],
                win.at[:, pl.ds(0, 2048)],
            )
            n = scan(jnp.int32(_WLAST), jnp.int32(_TLO))
            extract(n, jnp.int32(_WLAST), fg, win)

        @pl.when(t == 14)
        def _():
            pltpu.sync_copy(tailt.at[f], win.at[:, pl.ds(0, 128)])
            n = scan(jnp.int32(_TLO), jnp.int32(_V))
            extract(n, jnp.int32(_TAIL0), fg, win)

    def group(g, carry):
        field_of_group(g, 0)

        @pl.when(g < _NG - 1)
        def _():
            field_of_group(g, 1)

        plsc.subcore_barrier()
        @pl.when((t < 8) & ((g < _NG - 1) | (t < 4)))
        def _():
            off = pl.multiple_of(t * 8 * _B, 1024)
            goff = pl.multiple_of(
                (sc * _FPS + g * 2) * _D * _B + t * 8 * _B, 1024
            )
            pltpu.sync_copy(
                acc.at[pl.ds(off, 8 * _B)], out.at[pl.ds(goff, 8 * _B)]
            )

        plsc.subcore_barrier()
        return carry

    lax.fori_loop(0, _NG, group, 0)


@jax.jit
def kernel(indices, tables):
    idx4 = jnp.transpose(indices).reshape(_F, _CHK, 16)
    tab2 = jnp.transpose(tables, (0, 2, 1)).reshape(_F * _D, _V)
    tailt = jnp.transpose(tables[:, _TAIL0:, :], (0, 2, 1))

    out1 = pl.kernel(
        _body,
        out_type=jax.ShapeDtypeStruct((_F * _D * _B,), jnp.float32),
        mesh=plsc.VectorSubcoreMesh(core_axis_name="c", subcore_axis_name="s"),
        scratch_types=[
            pltpu.VMEM_SHARED((_GW,), jnp.float32),
            pltpu.VMEM((_D, _W), jnp.float32),
            pltpu.VMEM((_CHK, 16), jnp.int32),
            pltpu.VMEM((_B + 16,), jnp.int32),
            pltpu.VMEM((_B + 16,), jnp.int32),
            pltpu.VMEM((1024,), jnp.float32),
            pltpu.VMEM((1024,), jnp.int32),
        ],
        compiler_params=pltpu.CompilerParams(
            use_tc_tiling_on_sc=True, needs_layout_passes=False
        ),
    )(tab2, idx4, tailt)
    return out1.reshape(_F, _D, _B).transpose(2, 0, 1)


# speedup vs baseline: 29.9257x; 1.2227x over previous
import jax
import jax.numpy as jnp
from jax import lax
from jax.experimental import pallas as pl
from jax.experimental.pallas import tpu as pltpu
from jax.experimental.pallas import tpu_sc as plsc

_F = 26
_V = 100001
_D = 32
_B = 4096

_FPS = 13
_NG = 7
_W = 2176
_NFULL = 45
_WLAST = 45 * _W
_TLO = _WLAST + 2048
_TAIL0 = _V - 128
_CHK = _B // 16
_GW = 2 * _D * _B


def _body(tab, idx4, tailt, out, acc, win, idxv, cmp_v, cmp_p,
          st_v, st_a, dsem):
    sc = lax.axis_index("c")
    t = lax.axis_index("s")
    i16 = lax.iota(jnp.int32, 16)

    def bc(x):
        return jnp.full((16,), x, jnp.int32)

    def extract(ncur, base_col, fg, src_ref):
        def do_chunk(c, carry):
            mb0 = c * 32
            for g in range(2):
                mb = mb0 + g * 16
                vvec = cmp_v[pl.ds(mb, 16)]
                pvec = cmp_p[pl.ds(mb, 16)]
                valid = (bc(mb) + i16) < bc(ncur)
                col = jnp.where(valid, vvec - bc(base_col), bc(0))
                for d in range(32):
                    vals = plsc.load_gather(
                        src_ref, [jnp.full((16,), d, jnp.int32), col]
                    )
                    st_v[pl.ds(d * 32 + g * 16, 16)] = vals
                    addr = bc((fg * _D + d) * _B) + pvec
                    addr = jnp.where(
                        valid, addr, jnp.full((16,), -1, jnp.int32)
                    )
                    st_a[pl.ds(d * 32 + g * 16, 16)] = addr
            pltpu.sync_copy(
                st_v, acc.at[plsc.Indices(st_a, ignored_value=-1)]
            )
            return carry

        lax.fori_loop(0, (ncur + 31) // 32, do_chunk, 0)

    def scan(lo, hi):
        def chunk(i4, cur0):
            cur = cur0
            for u in range(4):
                i = i4 * 4 + u
                ids = idxv[i, :]
                m = (ids >= bc(lo)) & (ids < bc(hi))
                cnt = jnp.sum(m.astype(jnp.int32))
                curk = cur

                @pl.when(cnt > 0)
                def _(ids=ids, m=m, i=i, curk=curk):
                    pos = bc(i * 16) + i16
                    plsc.store_compressed(
                        cmp_v.at[pl.ds(curk, 16)], ids, mask=m)
                    plsc.store_compressed(
                        cmp_p.at[pl.ds(curk, 16)], pos, mask=m)

                cur = cur + cnt
            return cur

        return lax.fori_loop(0, _CHK // 4, chunk, 0)

    def field_of_group(g, fg):
        fl = g * 2 + fg
        f = sc * _FPS + fl
        rowb = pl.multiple_of(f * _D, 8)
        pltpu.sync_copy(idx4.at[f], idxv)

        for h in range(2):
            c0 = pl.multiple_of((h * 16 + t) * _W, 128)
            cp = pltpu.async_copy(
                tab.at[pl.ds(rowb, _D), pl.ds(c0, _W)], win, dsem)
            n = scan(c0, c0 + _W)
            cp.wait()
            extract(n, c0, fg, win)

        @pl.when(t < 13)
        def _():
            c2 = pl.multiple_of((32 + t) * _W, 128)
            cp = pltpu.async_copy(
                tab.at[pl.ds(rowb, _D), pl.ds(c2, _W)], win, dsem)
            n = scan(c2, c2 + _W)
            cp.wait()
            extract(n, c2, fg, win)

        @pl.when(t == 13)
        def _():
            pltpu.sync_copy(
                tab.at[pl.ds(rowb, _D), pl.ds(_WLAST, 2048)],
                win.at[:, pl.ds(0, 2048)],
            )
            n = scan(jnp.int32(_WLAST), jnp.int32(_TLO))
            extract(n, jnp.int32(_WLAST), fg, win)

        @pl.when(t == 14)
        def _():
            pltpu.sync_copy(tailt.at[f], win.at[:, pl.ds(0, 128)])
            n = scan(jnp.int32(_TLO), jnp.int32(_V))
            extract(n, jnp.int32(_TAIL0), fg, win)

    def group(g, carry):
        field_of_group(g, 0)

        @pl.when(g < _NG - 1)
        def _():
            field_of_group(g, 1)

        plsc.subcore_barrier()
        @pl.when((t < 8) & ((g < _NG - 1) | (t < 4)))
        def _():
            off = pl.multiple_of(t * 8 * _B, 1024)
            goff = pl.multiple_of(
                (sc * _FPS + g * 2) * _D * _B + t * 8 * _B, 1024
            )
            pltpu.sync_copy(
                acc.at[pl.ds(off, 8 * _B)], out.at[pl.ds(goff, 8 * _B)]
            )

        plsc.subcore_barrier()
        return carry

    lax.fori_loop(0, _NG, group, 0)


@jax.jit
def kernel(indices, tables):
    idx4 = jnp.transpose(indices).reshape(_F, _CHK, 16)
    tab2 = jnp.transpose(tables, (0, 2, 1)).reshape(_F * _D, _V)
    tailt = jnp.transpose(tables[:, _TAIL0:, :], (0, 2, 1))

    out1 = pl.kernel(
        _body,
        out_type=jax.ShapeDtypeStruct((_F * _D * _B,), jnp.float32),
        mesh=plsc.VectorSubcoreMesh(core_axis_name="c", subcore_axis_name="s"),
        scratch_types=[
            pltpu.VMEM_SHARED((_GW,), jnp.float32),
            pltpu.VMEM((_D, _W), jnp.float32),
            pltpu.VMEM((_CHK, 16), jnp.int32),
            pltpu.VMEM((_B + 16,), jnp.int32),
            pltpu.VMEM((_B + 16,), jnp.int32),
            pltpu.VMEM((1024,), jnp.float32),
            pltpu.VMEM((1024,), jnp.int32),
            pltpu.SemaphoreType.DMA,
        ],
        compiler_params=pltpu.CompilerParams(
            use_tc_tiling_on_sc=True, needs_layout_passes=False
        ),
    )(tab2, idx4, tailt)
    return out1.reshape(_F, _D, _B).transpose(2, 0, 1)


# speedup vs baseline: 32.8375x; 1.0973x over previous
import jax
import jax.numpy as jnp
from jax import lax
from jax.experimental import pallas as pl
from jax.experimental.pallas import tpu as pltpu
from jax.experimental.pallas import tpu_sc as plsc

_F = 26
_V = 100001
_D = 32
_B = 4096

_FPS = 13
_NG = 7
_W = 2176
_NFULL = 45
_WLAST = 45 * _W
_TLO = _WLAST + 2048
_TAIL0 = _V - 128
_CHK = _B // 16
_GW = 2 * _D * _B


def _body(tab, idx4, tailt, out, acc, win, idxv, cmp_v, cmp_p,
          st_v, st_a, dsem):
    sc = lax.axis_index("c")
    t = lax.axis_index("s")
    i16 = lax.iota(jnp.int32, 16)

    def bc(x):
        return jnp.full((16,), x, jnp.int32)

    def extract(ncur, base_col, fg, src_ref):
        def do_chunk(c, carry):
            mb0 = c * 32
            for g in range(2):
                mb = mb0 + g * 16
                vvec = cmp_v[pl.ds(mb, 16)]
                pvec = cmp_p[pl.ds(mb, 16)]
                valid = (bc(mb) + i16) < bc(ncur)
                col = jnp.where(valid, vvec - bc(base_col), bc(0))
                for d in range(32):
                    vals = plsc.load_gather(
                        src_ref, [jnp.full((16,), d, jnp.int32), col]
                    )
                    st_v[pl.ds(d * 32 + g * 16, 16)] = vals
                    addr = bc((fg * _D + d) * _B) + pvec
                    addr = jnp.where(
                        valid, addr, jnp.full((16,), -1, jnp.int32)
                    )
                    st_a[pl.ds(d * 32 + g * 16, 16)] = addr
            pltpu.sync_copy(
                st_v, acc.at[plsc.Indices(st_a, ignored_value=-1)]
            )
            return carry

        lax.fori_loop(0, (ncur + 31) // 32, do_chunk, 0)

    def scan(lo, hi):
        def chunk(i4, cur0):
            cur = cur0
            for u in range(4):
                i = i4 * 4 + u
                ids = idxv[i, :]
                m = (ids >= bc(lo)) & (ids < bc(hi))
                cnt = plsc.all_reduce_population_count(m)[0]
                curk = cur

                @pl.when(cnt > 0)
                def _(ids=ids, m=m, i=i, curk=curk):
                    pos = bc(i * 16) + i16
                    plsc.store_compressed(
                        cmp_v.at[pl.ds(curk, 16)], ids, mask=m)
                    plsc.store_compressed(
                        cmp_p.at[pl.ds(curk, 16)], pos, mask=m)

                cur = cur + cnt
            return cur

        return lax.fori_loop(0, _CHK // 4, chunk, 0)

    def field_of_group(g, fg):
        fl = g * 2 + fg
        f = sc * _FPS + fl
        rowb = pl.multiple_of(f * _D, 8)
        pltpu.sync_copy(idx4.at[f], idxv)

        for h in range(2):
            c0 = pl.multiple_of((h * 16 + t) * _W, 128)
            cp = pltpu.async_copy(
                tab.at[pl.ds(rowb, _D), pl.ds(c0, _W)], win, dsem)
            n = scan(c0, c0 + _W)
            cp.wait()
            extract(n, c0, fg, win)

        @pl.when(t < 13)
        def _():
            c2 = pl.multiple_of((32 + t) * _W, 128)
            cp = pltpu.async_copy(
                tab.at[pl.ds(rowb, _D), pl.ds(c2, _W)], win, dsem)
            n = scan(c2, c2 + _W)
            cp.wait()
            extract(n, c2, fg, win)

        @pl.when(t == 13)
        def _():
            pltpu.sync_copy(
                tab.at[pl.ds(rowb, _D), pl.ds(_WLAST, 2048)],
                win.at[:, pl.ds(0, 2048)],
            )
            n = scan(jnp.int32(_WLAST), jnp.int32(_TLO))
            extract(n, jnp.int32(_WLAST), fg, win)

        @pl.when(t == 14)
        def _():
            pltpu.sync_copy(tailt.at[f], win.at[:, pl.ds(0, 128)])
            n = scan(jnp.int32(_TLO), jnp.int32(_V))
            extract(n, jnp.int32(_TAIL0), fg, win)

    def group(g, carry):
        field_of_group(g, 0)

        @pl.when(g < _NG - 1)
        def _():
            field_of_group(g, 1)

        plsc.subcore_barrier()
        @pl.when((t < 8) & ((g < _NG - 1) | (t < 4)))
        def _():
            off = pl.multiple_of(t * 8 * _B, 1024)
            goff = pl.multiple_of(
                (sc * _FPS + g * 2) * _D * _B + t * 8 * _B, 1024
            )
            pltpu.sync_copy(
                acc.at[pl.ds(off, 8 * _B)], out.at[pl.ds(goff, 8 * _B)]
            )

        plsc.subcore_barrier()
        return carry

    lax.fori_loop(0, _NG, group, 0)


@jax.jit
def kernel(indices, tables):
    idx4 = jnp.transpose(indices).reshape(_F, _CHK, 16)
    tab2 = jnp.transpose(tables, (0, 2, 1)).reshape(_F * _D, _V)
    tailt = jnp.transpose(tables[:, _TAIL0:, :], (0, 2, 1))

    out1 = pl.kernel(
        _body,
        out_type=jax.ShapeDtypeStruct((_F * _D * _B,), jnp.float32),
        mesh=plsc.VectorSubcoreMesh(core_axis_name="c", subcore_axis_name="s"),
        scratch_types=[
            pltpu.VMEM_SHARED((_GW,), jnp.float32),
            pltpu.VMEM((_D, _W), jnp.float32),
            pltpu.VMEM((_CHK, 16), jnp.int32),
            pltpu.VMEM((_B + 16,), jnp.int32),
            pltpu.VMEM((_B + 16,), jnp.int32),
            pltpu.VMEM((1024,), jnp.float32),
            pltpu.VMEM((1024,), jnp.int32),
            pltpu.SemaphoreType.DMA,
        ],
        compiler_params=pltpu.CompilerParams(
            use_tc_tiling_on_sc=True, needs_layout_passes=False
        ),
    )(tab2, idx4, tailt)
    return out1.reshape(_F, _D, _B).transpose(2, 0, 1)
